# baseline (speedup 1.0000x reference)
import jax
import jax.numpy as jnp
from jax import lax
from jax.experimental import pallas as pl
from jax.experimental.pallas import tpu as pltpu
from jax.experimental.pallas import tpu_sc as plsc

_TAU = 0.6
_EPS = 1e-06

_NC = 2
_NS = 16
_NW = _NC * _NS
_NBINS = 32
_C = 19


def _dense_body(logits_ref, yhat_ref, conf_ref, idxm_ref):
    C, ROWS, W = logits_ref.shape[1], logits_ref.shape[2], logits_ref.shape[3]
    m = logits_ref[0, 0]
    yv = jnp.zeros((ROWS, W), jnp.int32)
    for c in range(1, C):
        xc = logits_ref[0, c]
        gt = xc > m
        m = jnp.where(gt, xc, m)
        yv = jnp.where(gt, c, yv)
    s = jnp.exp(logits_ref[0, 0] - m)
    for c in range(1, C):
        s = s + jnp.exp(logits_ref[0, c] - m)
    conf = 1.0 / s
    yhat_ref[0] = yv
    conf_ref[0] = conf
    im = jnp.where(conf > _TAU, yv, C)
    q = ROWS // 4
    idxm_ref[0] = (
        im[0:q]
        | (im[q : 2 * q] << 8)
        | (im[2 * q : 3 * q] << 16)
        | (im[3 * q : 4 * q] << 24)
    )


def _sc_hist_body(idxm_hbm, out_hbm, chunk_a, chunk_b, lane_hist_v, hist_v, sem_a, sem_b):
    core = lax.axis_index("c")
    sub = lax.axis_index("s")
    wid = sub * _NC + core
    EWORDS = idxm_hbm.shape[0]
    EW = EWORDS // _NW
    CH = 2048
    z16 = jnp.zeros((16,), jnp.float32)
    for r in range(16):
        lane_hist_v[r, pl.ds(0, 16)] = z16
        lane_hist_v[r, pl.ds(16, 16)] = z16
    ones16 = jnp.ones((16,), jnp.float32)
    lanes16 = lax.iota(jnp.int32, 16)

    base = wid * EW
    NCK = EW // CH
    bufs = [chunk_a, chunk_b]
    sems = [sem_a, sem_b]
    handles = [None, None]
    handles[0] = pltpu.async_copy(idxm_hbm.at[pl.ds(base, CH)], chunk_a, sem_a)
    for ck in range(NCK):
        cur = bufs[ck % 2]
        if ck + 1 < NCK:
            handles[(ck + 1) % 2] = pltpu.async_copy(
                idxm_hbm.at[pl.ds(base + (ck + 1) * CH, CH)],
                bufs[(ck + 1) % 2],
                sems[(ck + 1) % 2],
            )
        handles[ck % 2].wait()

        def inner(j, carry):
            w0 = cur[pl.ds(j * 32, 16)]
            w1 = cur[pl.ds(j * 32 + 16, 16)]
            d0 = (w0 - 0x13131313) & (~w0) & jnp.int32(-2139062144)
            d1 = (w1 - 0x13131313) & (~w1) & jnp.int32(-2139062144)
            nz = jnp.sum(jnp.where((d0 | d1) != 0, 1, 0).astype(jnp.int32))

            @pl.when(nz > 0)
            def _count():
                for w in (w0, w1):
                    for k in range(4):
                        idx = lax.shift_right_logical(w, 8 * k) & 0xFF
                        plsc.addupdate_scatter(
                            lane_hist_v, [lanes16, idx], ones16, mask=idx < _C
                        )

            return carry

        lax.fori_loop(0, CH // 32, inner, 0)

    lo = lane_hist_v[0, pl.ds(0, 16)]
    hi = lane_hist_v[0, pl.ds(16, 16)]
    for r in range(1, 16):
        lo = lo + lane_hist_v[r, pl.ds(0, 16)]
        hi = hi + lane_hist_v[r, pl.ds(16, 16)]
    hist_v[pl.ds(0, 16)] = lo
    hist_v[pl.ds(16, 16)] = hi
    pltpu.sync_copy(hist_v, out_hbm.at[wid])


def _final_body(sigma_ref, conf_ref, yhat_ref, delta_ref, tc_ref):
    i = pl.program_id(0)
    C = tc_ref.shape[1]
    ROWS, W = conf_ref.shape[1], conf_ref.shape[2]
    sigma = jnp.sum(sigma_ref[...], axis=0, keepdims=True)
    sig19 = sigma[:, :C]
    mx = jnp.maximum(jnp.max(sig19), _EPS)
    sh = sig19 / mx
    tc = sh / (2.0 - jnp.minimum(sh, 1.0)) * _TAU

    @pl.when(i == 0)
    def _write_tc():
        tc_ref[...] = tc

    yv = yhat_ref[0]
    conf = conf_ref[0]
    tmap = jnp.broadcast_to(tc[0:1, 0:1], (ROWS, W))
    for c in range(1, C):
        tmap = jnp.where(yv == c, jnp.broadcast_to(tc[0:1, c : c + 1], (ROWS, W)), tmap)
    delta_ref[0] = conf > tmap


def kernel(logits):
    B, C, H, W = logits.shape
    ROWS = 256
    NB = H // ROWS
    N = B * NB

    yhat, conf, idxm = pl.pallas_call(
        _dense_body,
        grid=(N,),
        in_specs=[pl.BlockSpec((1, C, ROWS, W), lambda i: (i // NB, 0, i % NB, 0))],
        out_specs=[
            pl.BlockSpec((1, ROWS, W), lambda i: (i // NB, i % NB, 0)),
            pl.BlockSpec((1, ROWS, W), lambda i: (i // NB, i % NB, 0)),
            pl.BlockSpec((1, ROWS // 4, W), lambda i: (i // NB, i % NB, 0)),
        ],
        out_shape=[
            jax.ShapeDtypeStruct((B, H, W), jnp.int32),
            jax.ShapeDtypeStruct((B, H, W), jnp.float32),
            jax.ShapeDtypeStruct((B, H // 4, W), jnp.int32),
        ],
    )(logits)

    mesh = plsc.VectorSubcoreMesh(core_axis_name="c", subcore_axis_name="s")
    sigma = pl.kernel(
        _sc_hist_body,
        out_type=jax.ShapeDtypeStruct((_NW, _NBINS), jnp.float32),
        mesh=mesh,
        compiler_params=pltpu.CompilerParams(needs_layout_passes=False),
        scratch_types=[
            pltpu.VMEM((2048,), jnp.int32),
            pltpu.VMEM((2048,), jnp.int32),
            pltpu.VMEM((_NS, _NBINS), jnp.float32),
            pltpu.VMEM((_NBINS,), jnp.float32),
            pltpu.SemaphoreType.DMA,
            pltpu.SemaphoreType.DMA,
        ],
    )(idxm.reshape(-1))

    delta, tc = pl.pallas_call(
        _final_body,
        grid=(N,),
        in_specs=[
            pl.BlockSpec((_NW, _NBINS), lambda i: (0, 0)),
            pl.BlockSpec((1, ROWS, W), lambda i: (i // NB, i % NB, 0)),
            pl.BlockSpec((1, ROWS, W), lambda i: (i // NB, i % NB, 0)),
        ],
        out_specs=[
            pl.BlockSpec((1, ROWS, W), lambda i: (i // NB, i % NB, 0)),
            pl.BlockSpec((1, C), lambda i: (0, 0)),
        ],
        out_shape=[
            jax.ShapeDtypeStruct((B, H, W), jnp.bool_),
            jax.ShapeDtypeStruct((1, C), jnp.float32),
        ],
    )(sigma, conf, yhat)
    return (delta, tc.reshape(C), yhat)


# speedup vs baseline: 1.0454x; 1.0454x over previous
import jax
import jax.numpy as jnp
from jax import lax
from jax.experimental import pallas as pl
from jax.experimental.pallas import tpu as pltpu
from jax.experimental.pallas import tpu_sc as plsc

_TAU = 0.6
_EPS = 1e-06

_NC = 2
_NS = 16
_NW = _NC * _NS
_NBINS = 32
_C = 19


def _dense_body(logits_ref, yhat_ref, conf_ref, idxm_ref):
    C, ROWS, W = logits_ref.shape[1], logits_ref.shape[2], logits_ref.shape[3]
    m = logits_ref[0, 0]
    yv = jnp.zeros((ROWS, W), jnp.int32)
    for c in range(1, C):
        xc = logits_ref[0, c]
        gt = xc > m
        m = jnp.where(gt, xc, m)
        yv = jnp.where(gt, c, yv)
    s = jnp.exp(logits_ref[0, 0] - m)
    for c in range(1, C):
        s = s + jnp.exp(logits_ref[0, c] - m)
    conf = 1.0 / s
    yhat_ref[0] = yv
    conf_ref[0] = conf
    im = jnp.where(conf > _TAU, yv, C)
    q = ROWS // 4
    idxm_ref[0] = (
        im[0:q]
        | (im[q : 2 * q] << 8)
        | (im[2 * q : 3 * q] << 16)
        | (im[3 * q : 4 * q] << 24)
    )


def _sc_hist_body(idxm_hbm, out_hbm, chunk_a, chunk_b, lane_hist_v, hist_v, sem_a, sem_b):
    core = lax.axis_index("c")
    sub = lax.axis_index("s")
    wid = sub * _NC + core
    EWORDS = idxm_hbm.shape[0]
    EW = EWORDS // _NW
    CH = 2048
    z16 = jnp.zeros((16,), jnp.float32)
    for r in range(16):
        lane_hist_v[r, pl.ds(0, 16)] = z16
        lane_hist_v[r, pl.ds(16, 16)] = z16
    ones16 = jnp.ones((16,), jnp.float32)
    lanes16 = lax.iota(jnp.int32, 16)

    base = wid * EW
    NCK = EW // CH
    bufs = [chunk_a, chunk_b]
    sems = [sem_a, sem_b]
    handles = [None, None]
    handles[0] = pltpu.async_copy(idxm_hbm.at[pl.ds(base, CH)], chunk_a, sem_a)
    for ck in range(NCK):
        cur = bufs[ck % 2]
        if ck + 1 < NCK:
            handles[(ck + 1) % 2] = pltpu.async_copy(
                idxm_hbm.at[pl.ds(base + (ck + 1) * CH, CH)],
                bufs[(ck + 1) % 2],
                sems[(ck + 1) % 2],
            )
        handles[ck % 2].wait()

        def inner(j, carry):
            pairs = []
            for p in range(2):
                w0 = cur[pl.ds(j * 64 + p * 32, 16)]
                w1 = cur[pl.ds(j * 64 + p * 32 + 16, 16)]
                d0 = (w0 - 0x13131313) & (~w0) & jnp.int32(-2139062144)
                d1 = (w1 - 0x13131313) & (~w1) & jnp.int32(-2139062144)
                nz = jnp.sum(jnp.where((d0 | d1) != 0, 1, 0).astype(jnp.int32))
                pairs.append((nz, w0, w1))

            for nz, w0, w1 in pairs:

                @pl.when(nz > 0)
                def _count(w0=w0, w1=w1):
                    for w in (w0, w1):
                        for k in range(4):
                            idx = lax.shift_right_logical(w, 8 * k) & 0xFF
                            plsc.addupdate_scatter(
                                lane_hist_v, [lanes16, idx], ones16, mask=idx < _C
                            )

            return carry

        lax.fori_loop(0, CH // 64, inner, 0)

    lo = lane_hist_v[0, pl.ds(0, 16)]
    hi = lane_hist_v[0, pl.ds(16, 16)]
    for r in range(1, 16):
        lo = lo + lane_hist_v[r, pl.ds(0, 16)]
        hi = hi + lane_hist_v[r, pl.ds(16, 16)]
    hist_v[pl.ds(0, 16)] = lo
    hist_v[pl.ds(16, 16)] = hi
    pltpu.sync_copy(hist_v, out_hbm.at[wid])


def _final_body(sigma_ref, conf_ref, yhat_ref, delta_ref, tc_ref):
    i = pl.program_id(0)
    C = tc_ref.shape[1]
    ROWS, W = conf_ref.shape[1], conf_ref.shape[2]
    sigma = jnp.sum(sigma_ref[...], axis=0, keepdims=True)
    sig19 = sigma[:, :C]
    mx = jnp.maximum(jnp.max(sig19), _EPS)
    sh = sig19 / mx
    tc = sh / (2.0 - jnp.minimum(sh, 1.0)) * _TAU

    @pl.when(i == 0)
    def _write_tc():
        tc_ref[...] = tc

    yv = yhat_ref[0]
    conf = conf_ref[0]
    tmap = jnp.broadcast_to(tc[0:1, 0:1], (ROWS, W))
    for c in range(1, C):
        tmap = jnp.where(yv == c, jnp.broadcast_to(tc[0:1, c : c + 1], (ROWS, W)), tmap)
    delta_ref[0] = conf > tmap


def kernel(logits):
    B, C, H, W = logits.shape
    ROWS = 256
    NB = H // ROWS
    N = B * NB

    yhat, conf, idxm = pl.pallas_call(
        _dense_body,
        grid=(N,),
        in_specs=[pl.BlockSpec((1, C, ROWS, W), lambda i: (i // NB, 0, i % NB, 0))],
        out_specs=[
            pl.BlockSpec((1, ROWS, W), lambda i: (i // NB, i % NB, 0)),
            pl.BlockSpec((1, ROWS, W), lambda i: (i // NB, i % NB, 0)),
            pl.BlockSpec((1, ROWS // 4, W), lambda i: (i // NB, i % NB, 0)),
        ],
        out_shape=[
            jax.ShapeDtypeStruct((B, H, W), jnp.int32),
            jax.ShapeDtypeStruct((B, H, W), jnp.float32),
            jax.ShapeDtypeStruct((B, H // 4, W), jnp.int32),
        ],
    )(logits)

    mesh = plsc.VectorSubcoreMesh(core_axis_name="c", subcore_axis_name="s")
    sigma = pl.kernel(
        _sc_hist_body,
        out_type=jax.ShapeDtypeStruct((_NW, _NBINS), jnp.float32),
        mesh=mesh,
        compiler_params=pltpu.CompilerParams(needs_layout_passes=False),
        scratch_types=[
            pltpu.VMEM((2048,), jnp.int32),
            pltpu.VMEM((2048,), jnp.int32),
            pltpu.VMEM((_NS, _NBINS), jnp.float32),
            pltpu.VMEM((_NBINS,), jnp.float32),
            pltpu.SemaphoreType.DMA,
            pltpu.SemaphoreType.DMA,
        ],
    )(idxm.reshape(-1))

    delta, tc = pl.pallas_call(
        _final_body,
        grid=(N,),
        in_specs=[
            pl.BlockSpec((_NW, _NBINS), lambda i: (0, 0)),
            pl.BlockSpec((1, ROWS, W), lambda i: (i // NB, i % NB, 0)),
            pl.BlockSpec((1, ROWS, W), lambda i: (i // NB, i % NB, 0)),
        ],
        out_specs=[
            pl.BlockSpec((1, ROWS, W), lambda i: (i // NB, i % NB, 0)),
            pl.BlockSpec((1, C), lambda i: (0, 0)),
        ],
        out_shape=[
            jax.ShapeDtypeStruct((B, H, W), jnp.bool_),
            jax.ShapeDtypeStruct((1, C), jnp.float32),
        ],
    )(sigma, conf, yhat)
    return (delta, tc.reshape(C), yhat)


# speedup vs baseline: 1.0731x; 1.0264x over previous
import jax
import jax.numpy as jnp
from jax import lax
from jax.experimental import pallas as pl
from jax.experimental.pallas import tpu as pltpu
from jax.experimental.pallas import tpu_sc as plsc

_TAU = 0.6
_EPS = 1e-06

_NC = 2
_NS = 16
_NW = _NC * _NS
_NBINS = 32
_C = 19


def _dense_body(logits_ref, yhat_ref, conf_ref, idxm_ref):
    C, ROWS, W = logits_ref.shape[1], logits_ref.shape[2], logits_ref.shape[3]
    m = logits_ref[0, 0]
    yv = jnp.zeros((ROWS, W), jnp.int32)
    for c in range(1, C):
        xc = logits_ref[0, c]
        gt = xc > m
        m = jnp.where(gt, xc, m)
        yv = jnp.where(gt, c, yv)
    s = jnp.exp(logits_ref[0, 0] - m)
    for c in range(1, C):
        s = s + jnp.exp(logits_ref[0, c] - m)
    conf = 1.0 / s
    yhat_ref[0] = yv
    conf_ref[0] = conf
    im = jnp.where(conf > _TAU, yv, C)
    q = ROWS // 4
    idxm_ref[0] = (
        im[0:q]
        | (im[q : 2 * q] << 8)
        | (im[2 * q : 3 * q] << 16)
        | (im[3 * q : 4 * q] << 24)
    )


def _sc_hist_body(idxm_hbm, out_hbm, chunk_a, chunk_b, lane_hist_v, hist_v, sem_a, sem_b):
    core = lax.axis_index("c")
    sub = lax.axis_index("s")
    wid = sub * _NC + core
    EWORDS = idxm_hbm.shape[0]
    EW = EWORDS // _NW
    CH = 2048
    z16 = jnp.zeros((16,), jnp.float32)
    for r in range(16):
        lane_hist_v[r, pl.ds(0, 16)] = z16
        lane_hist_v[r, pl.ds(16, 16)] = z16
    ones16 = jnp.ones((16,), jnp.float32)
    lanes16 = lax.iota(jnp.int32, 16)

    base = wid * EW
    NCK = EW // CH
    bufs = [chunk_a, chunk_b]
    sems = [sem_a, sem_b]
    handles = [None, None]
    handles[0] = pltpu.async_copy(idxm_hbm.at[pl.ds(base, CH)], chunk_a, sem_a)
    for ck in range(NCK):
        cur = bufs[ck % 2]
        if ck + 1 < NCK:
            handles[(ck + 1) % 2] = pltpu.async_copy(
                idxm_hbm.at[pl.ds(base + (ck + 1) * CH, CH)],
                bufs[(ck + 1) % 2],
                sems[(ck + 1) % 2],
            )
        handles[ck % 2].wait()

        def inner(j, carry):
            pairs = []
            for p in range(4):
                w0 = cur[pl.ds(j * 128 + p * 32, 16)]
                w1 = cur[pl.ds(j * 128 + p * 32 + 16, 16)]
                d0 = (w0 - 0x13131313) & (~w0) & jnp.int32(-2139062144)
                d1 = (w1 - 0x13131313) & (~w1) & jnp.int32(-2139062144)
                nz = jnp.sum(jnp.where((d0 | d1) != 0, 1, 0).astype(jnp.int32))
                pairs.append((nz, w0, w1))

            for nz, w0, w1 in pairs:

                @pl.when(nz > 0)
                def _count(w0=w0, w1=w1):
                    for w in (w0, w1):
                        for k in range(4):
                            idx = lax.shift_right_logical(w, 8 * k) & 0xFF
                            plsc.addupdate_scatter(
                                lane_hist_v, [lanes16, idx], ones16, mask=idx < _C
                            )

            return carry

        lax.fori_loop(0, CH // 128, inner, 0)

    lo = lane_hist_v[0, pl.ds(0, 16)]
    hi = lane_hist_v[0, pl.ds(16, 16)]
    for r in range(1, 16):
        lo = lo + lane_hist_v[r, pl.ds(0, 16)]
        hi = hi + lane_hist_v[r, pl.ds(16, 16)]
    hist_v[pl.ds(0, 16)] = lo
    hist_v[pl.ds(16, 16)] = hi
    pltpu.sync_copy(hist_v, out_hbm.at[wid])


def _final_body(sigma_ref, conf_ref, yhat_ref, delta_ref, tc_ref):
    i = pl.program_id(0)
    C = tc_ref.shape[1]
    ROWS, W = conf_ref.shape[1], conf_ref.shape[2]
    sigma = jnp.sum(sigma_ref[...], axis=0, keepdims=True)
    sig19 = sigma[:, :C]
    mx = jnp.maximum(jnp.max(sig19), _EPS)
    sh = sig19 / mx
    tc = sh / (2.0 - jnp.minimum(sh, 1.0)) * _TAU

    @pl.when(i == 0)
    def _write_tc():
        tc_ref[...] = tc

    yv = yhat_ref[0]
    conf = conf_ref[0]
    tmap = jnp.broadcast_to(tc[0:1, 0:1], (ROWS, W))
    for c in range(1, C):
        tmap = jnp.where(yv == c, jnp.broadcast_to(tc[0:1, c : c + 1], (ROWS, W)), tmap)
    delta_ref[0] = conf > tmap


def kernel(logits):
    B, C, H, W = logits.shape
    ROWS = 256
    NB = H // ROWS
    N = B * NB

    yhat, conf, idxm = pl.pallas_call(
        _dense_body,
        grid=(N,),
        in_specs=[pl.BlockSpec((1, C, ROWS, W), lambda i: (i // NB, 0, i % NB, 0))],
        out_specs=[
            pl.BlockSpec((1, ROWS, W), lambda i: (i // NB, i % NB, 0)),
            pl.BlockSpec((1, ROWS, W), lambda i: (i // NB, i % NB, 0)),
            pl.BlockSpec((1, ROWS // 4, W), lambda i: (i // NB, i % NB, 0)),
        ],
        out_shape=[
            jax.ShapeDtypeStruct((B, H, W), jnp.int32),
            jax.ShapeDtypeStruct((B, H, W), jnp.float32),
            jax.ShapeDtypeStruct((B, H // 4, W), jnp.int32),
        ],
    )(logits)

    mesh = plsc.VectorSubcoreMesh(core_axis_name="c", subcore_axis_name="s")
    sigma = pl.kernel(
        _sc_hist_body,
        out_type=jax.ShapeDtypeStruct((_NW, _NBINS), jnp.float32),
        mesh=mesh,
        compiler_params=pltpu.CompilerParams(needs_layout_passes=False),
        scratch_types=[
            pltpu.VMEM((2048,), jnp.int32),
            pltpu.VMEM((2048,), jnp.int32),
            pltpu.VMEM((_NS, _NBINS), jnp.float32),
            pltpu.VMEM((_NBINS,), jnp.float32),
            pltpu.SemaphoreType.DMA,
            pltpu.SemaphoreType.DMA,
        ],
    )(idxm.reshape(-1))

    delta, tc = pl.pallas_call(
        _final_body,
        grid=(N,),
        in_specs=[
            pl.BlockSpec((_NW, _NBINS), lambda i: (0, 0)),
            pl.BlockSpec((1, ROWS, W), lambda i: (i // NB, i % NB, 0)),
            pl.BlockSpec((1, ROWS, W), lambda i: (i // NB, i % NB, 0)),
        ],
        out_specs=[
            pl.BlockSpec((1, ROWS, W), lambda i: (i // NB, i % NB, 0)),
            pl.BlockSpec((1, C), lambda i: (0, 0)),
        ],
        out_shape=[
            jax.ShapeDtypeStruct((B, H, W), jnp.bool_),
            jax.ShapeDtypeStruct((1, C), jnp.float32),
        ],
    )(sigma, conf, yhat)
    return (delta, tc.reshape(C), yhat)
